# baseline (speedup 1.0000x reference)
import functools

import jax
import jax.numpy as jnp
from jax import lax
from jax.experimental import pallas as pl
from jax.experimental.pallas import tpu as pltpu
from jax.experimental.pallas import tpu_sc as plsc

_INPUT_DIM = 30
_N_CAT = 26
_VOCAB = 100000
_EMB = 16
_BATCH = 4096
_N_CONT = _INPUT_DIM - _N_CAT
_OUT_DIM = _N_CAT * _EMB + _N_CONT
_NC = 2
_NS = 16
_NW = _NC * _NS
_BPW = _BATCH // _NW
_L = 16
_N_IDX = _N_CAT * _BPW


def _body(x_hbm, tab_hbm, out_hbm, xb, idxb, embb, stage, gsem, wsem):
    c = lax.axis_index("c")
    s = lax.axis_index("s")
    w = s * _NC + c
    base = w * _BPW

    pltpu.sync_copy(x_hbm.at[pl.ds(base, _BPW), :], xb)

    lanes = lax.iota(jnp.int32, _L)

    def fire_f(f, carry):
        def chunk(i, carry2):
            r = i * _L + lanes
            col = jnp.zeros((_L,), jnp.int32) + f
            xv = plsc.load_gather(xb, [r, col])
            idxb[f, pl.ds(i * _L, _L)] = xv
            return carry2
        lax.fori_loop(0, _BPW // _L, chunk, 0)
        pltpu.async_copy(tab_hbm.at[f].at[idxb.at[f]],
                         embb.at[pl.ds(f * _BPW, _BPW)], gsem)
        return carry

    lax.fori_loop(0, _N_CAT, fire_f, 0)

    def cc_chunk(v, carry):
        vv = v * _L + lanes
        i = lax.shift_right_logical(vv, 2)
        j = lax.bitwise_and(vv, 3)
        xv = plsc.load_gather(xb, [i, j + _N_CAT])
        plsc.store_scatter(stage, [i, j + _N_CAT * _EMB], xv.astype(jnp.float32))
        return carry

    lax.fori_loop(0, (_BPW * _N_CONT) // _L, cc_chunk, 0)

    pltpu.make_async_copy(tab_hbm.at[0].at[pl.ds(0, _N_IDX), :], embb, gsem).wait()

    def repack_b(b, carry):
        for f in range(_N_CAT):
            stage[b, pl.ds(f * _EMB, _EMB)] = embb[f * _BPW + b, :]
        return carry

    lax.fori_loop(0, _BPW, repack_b, 0)

    pltpu.async_copy(stage, out_hbm.at[pl.ds(base, _BPW), :], wsem)
    pltpu.make_async_copy(out_hbm.at[pl.ds(base, _BPW), :], stage, wsem).wait()


_emb_call = functools.partial(
    pl.kernel,
    mesh=plsc.VectorSubcoreMesh(core_axis_name="c", subcore_axis_name="s"),
    out_type=jax.ShapeDtypeStruct((_BATCH, _OUT_DIM), jnp.float32),
    compiler_params=pltpu.CompilerParams(needs_layout_passes=False,
                                         use_tc_tiling_on_sc=False),
    scratch_types=[
        pltpu.VMEM((_BPW, _INPUT_DIM), jnp.int32),
        pltpu.VMEM((_N_CAT, _BPW), jnp.int32),
        pltpu.VMEM((_N_IDX, _EMB), jnp.float32),
        pltpu.VMEM((_BPW, _OUT_DIM), jnp.float32),
        pltpu.SemaphoreType.DMA,
        pltpu.SemaphoreType.DMA,
    ],
)(_body)


def kernel(x, tables):
    return _emb_call(x, tables)


# speedup vs baseline: 8.4274x; 8.4274x over previous
import functools

import jax
import jax.numpy as jnp
from jax import lax
from jax.experimental import pallas as pl
from jax.experimental.pallas import tpu as pltpu
from jax.experimental.pallas import tpu_sc as plsc

_INPUT_DIM = 30
_N_CAT = 26
_VOCAB = 100000
_EMB = 16
_BATCH = 4096
_N_CONT = _INPUT_DIM - _N_CAT
_TAB_ROWS = _N_CAT * _EMB
_OUT_ROWS = _TAB_ROWS + _N_CONT
_NC = 2
_NS = 16
_NW = _NC * _NS
_RPW = _TAB_ROWS // _NW
_L = 16


def _body(xt_hbm, tt_hbm, out_hbm, rowb, colb, outb):
    c = lax.axis_index("c")
    s = lax.axis_index("s")
    w = s * _NC + c

    def gather_row(_, carry):
        def chunk(i, cc):
            idx = colb[pl.ds(i * _L, _L)]
            outb[pl.ds(i * _L, _L)] = plsc.load_gather(rowb, [idx])
            return cc
        lax.fori_loop(0, _BATCH // _L, chunk, 0, unroll=8)
        return carry

    def do_r(j, carry):
        r = w * _RPW + j
        f = r // _EMB
        e = lax.rem(r, _EMB)
        pltpu.sync_copy(tt_hbm.at[f].at[e], rowb)
        pltpu.sync_copy(xt_hbm.at[f], colb)
        gather_row(None, None)
        pltpu.sync_copy(outb, out_hbm.at[r])
        return carry

    lax.fori_loop(0, _RPW, do_r, 0)

    @pl.when(w < _N_CONT)
    def _():
        pltpu.sync_copy(xt_hbm.at[_N_CAT + w], colb)

        def chunk(i, cc):
            outb[pl.ds(i * _L, _L)] = colb[pl.ds(i * _L, _L)].astype(jnp.float32)
            return cc
        lax.fori_loop(0, _BATCH // _L, chunk, 0, unroll=8)
        pltpu.sync_copy(outb, out_hbm.at[_TAB_ROWS + w])


_emb_call = functools.partial(
    pl.kernel,
    mesh=plsc.VectorSubcoreMesh(core_axis_name="c", subcore_axis_name="s"),
    out_type=jax.ShapeDtypeStruct((_OUT_ROWS, _BATCH), jnp.float32),
    compiler_params=pltpu.CompilerParams(needs_layout_passes=False,
                                         use_tc_tiling_on_sc=True),
    scratch_types=[
        pltpu.VMEM((_VOCAB,), jnp.float32),
        pltpu.VMEM((_BATCH,), jnp.int32),
        pltpu.VMEM((_BATCH,), jnp.float32),
    ],
)(_body)


def kernel(x, tables):
    xt = x.T
    tt = tables.transpose(0, 2, 1)
    out_t = _emb_call(xt, tt)
    return out_t.T
